# baseline (speedup 1.0000x reference)
import functools

import jax
import jax.numpy as jnp
from jax import lax
from jax.experimental import pallas as pl
from jax.experimental.pallas import tpu as pltpu
from jax.experimental.pallas import tpu_sc as plsc

N0 = 100000
R = 16
B = 16384
NC = 2
NSUB = 16
NW = NC * NSUB
PER_TILE = B // NW
C = 32
NCHUNK = PER_TILE // C

_mesh = plsc.VectorSubcoreMesh(
    core_axis_name="c", subcore_axis_name="s", num_cores=NC, num_subcores=NSUB
)


JCH = 782


@functools.partial(
    pl.kernel,
    out_type=jax.ShapeDtypeStruct((N0, 128), jnp.float32),
    mesh=_mesh,
    compiler_params=pltpu.CompilerParams(
        needs_layout_passes=False, use_tc_tiling_on_sc=True,
        disable_bounds_checks=True),
    scratch_types=[
        pltpu.VMEM((2 * 16, 128), jnp.float32),
        pltpu.VMEM((2 * 16, 128), jnp.float32),
        pltpu.VMEM((2 * 128, 128), jnp.float32),
        pltpu.SemaphoreType.DMA,
        pltpu.SemaphoreType.DMA,
        pltpu.SemaphoreType.DMA,
        pltpu.SemaphoreType.DMA,
    ],
)
def _small_build(t0, t2, out, b0, b2, orows, sin0, sin1, sout0, sout1):
    wid = lax.axis_index("s") * NC + lax.axis_index("c")
    lane16 = lax.iota(jnp.int32, 16)
    NFULL = JCH - 1
    NIT = (NFULL + NW - 1) // NW
    sins = (sin0, sin1)
    souts = (sout0, sout1)

    def ch_of(it):
        return jnp.minimum(it * NW + wid, NFULL - 1)

    def fire(it):
        s = it % 2
        jb = ch_of(it) * 128
        return [
            pltpu.async_copy(t0.at[pl.ds(0, 16), pl.ds(jb, 128)],
                             b0.at[pl.ds(s * 16, 16)], sins[s]),
            pltpu.async_copy(t2.at[pl.ds(0, 16), pl.ds(jb, 128)],
                             b2.at[pl.ds(s * 16, 16)], sins[s]),
        ]

    def transpose_rows(s, nrows):
        def jrow(jl, c):
            jv = jnp.full((16,), jl, jnp.int32)
            orows[jl + s * 128, pl.ds(0, 16)] = plsc.load_gather(
                b0, [lane16 + s * 16, jv])
            orows[jl + s * 128, pl.ds(16, 16)] = plsc.load_gather(
                b2, [lane16 + s * 16, jv])
            return c
        lax.fori_loop(0, nrows, jrow, 0)

    pend = fire(0)
    outcps = []
    for it in range(NIT):
        s = it % 2
        nxt = fire(it + 1) if it + 1 < NIT else []
        for cp in pend:
            cp.wait()
        pend = nxt
        if len(outcps) == 2:
            outcps.pop(0).wait()
        transpose_rows(s, 128)
        outcps.append(pltpu.async_copy(
            orows.at[pl.ds(s * 128, 128)],
            out.at[pl.ds(ch_of(it) * 128, 128)], souts[s]))
    for cp in outcps:
        cp.wait()

    @pl.when(wid == NW - 1)
    def _tail():
        jb = pl.multiple_of(NFULL * 128 + (wid - (NW - 1)), 128)
        cp0 = pltpu.async_copy(t0.at[pl.ds(0, 16), pl.ds(jb, 128)],
                               b0.at[pl.ds(0, 16)], sin0)
        cp1 = pltpu.async_copy(t2.at[pl.ds(0, 16), pl.ds(jb, 128)],
                               b2.at[pl.ds(0, 16)], sin0)
        cp0.wait()
        cp1.wait()
        transpose_rows(0, 32)
        pltpu.async_copy(orows.at[pl.ds(0, 32)],
                         out.at[pl.ds(jb, 32)], sout0).wait()


@functools.partial(
    pl.kernel,
    out_type=jax.ShapeDtypeStruct((B,), jnp.float32),
    mesh=_mesh,
    compiler_params=pltpu.CompilerParams(
        needs_layout_passes=False, use_tc_tiling_on_sc=True),
    scratch_types=[
        pltpu.VMEM((3 * PER_TILE,), jnp.float32),
        pltpu.VMEM((3 * PER_TILE,), jnp.float32),
        pltpu.VMEM((PER_TILE,), jnp.int32),
        pltpu.VMEM((PER_TILE,), jnp.int32),
        pltpu.VMEM((PER_TILE,), jnp.int32),
        pltpu.VMEM((PER_TILE,), jnp.int32),
        pltpu.VMEM((PER_TILE,), jnp.int32),
        pltpu.VMEM((PER_TILE,), jnp.int32),
        pltpu.VMEM((2 * C, 128), jnp.float32),
        pltpu.VMEM((2 * C, 128), jnp.float32),
        pltpu.VMEM((2 * C, 128), jnp.float32),
        pltpu.VMEM((2 * C, 128), jnp.float32),
        pltpu.VMEM((2 * C, 256), jnp.float32),
        pltpu.VMEM((2 * C, 256), jnp.float32),
        pltpu.VMEM((2 * C,), jnp.float32),
        pltpu.SemaphoreType.DMA,
        pltpu.SemaphoreType.DMA,
        pltpu.SemaphoreType.DMA,
        pltpu.SemaphoreType.DMA,
    ],
)
def _tt_sc(xT, big, small, out, xbuf, wbuf, jlo0, jhi0, jlo1, jhi1,
           jlo2, jhi2, rows0lo, rows0hi, rows2lo, rows2hi,
           rows1lo, rows1hi, outv, sg0, sg1, so0, so1):
    sgs = (sg0, sg1)
    sos = (so0, so1)
    wid = lax.axis_index("s") * NC + lax.axis_index("c")
    base0 = wid * PER_TILE

    xcp = [pltpu.async_copy(xT.at[pl.ds(i * B + base0, PER_TILE)],
                            xbuf.at[pl.ds(i * PER_TILE, PER_TILE)], sg0)
           for i in range(3)]
    for cp in xcp:
        cp.wait()

    for i in range(3):
        jlo_ref = (jlo0, jlo1, jlo2)[i]
        jhi_ref = (jhi0, jhi1, jhi2)[i]
        for t in range(PER_TILE // 16):
            sl = pl.ds(t * 16, 16)
            xv = xbuf[pl.ds(i * PER_TILE + t * 16, 16)]
            xr = (xv + 1.0) * (0.5 * (N0 - 1))
            xr = jnp.minimum(jnp.maximum(xr, 0.0), float(N0 - 1))
            jlo = xr.astype(jnp.int32)
            w = xr - jlo.astype(jnp.float32)
            jhi = jnp.where(w > 0.0, jlo + 1, jlo)
            wbuf[pl.ds(i * PER_TILE + t * 16, 16)] = w
            jlo_ref[sl] = jlo
            jhi_ref[sl] = jhi

    def fire(k):
        sg = sgs[k % 2]
        ssl = pl.ds((k % 2) * C, C)
        ksl = pl.ds(k * C, C)
        return [
            pltpu.async_copy(small.at[jlo0.at[ksl]], rows0lo.at[ssl], sg),
            pltpu.async_copy(small.at[jhi0.at[ksl]], rows0hi.at[ssl], sg),
            pltpu.async_copy(small.at[jlo2.at[ksl]], rows2lo.at[ssl], sg),
            pltpu.async_copy(small.at[jhi2.at[ksl]], rows2hi.at[ssl], sg),
            pltpu.async_copy(big.at[jlo1.at[ksl]], rows1lo.at[ssl], sg),
            pltpu.async_copy(big.at[jhi1.at[ksl]], rows1hi.at[ssl], sg),
        ]

    lane = lax.iota(jnp.int32, 16)
    lane0 = lane == 0
    outcps = []
    pend = fire(0)
    for k in range(NCHUNK):
        s = k % 2
        nxt = fire(k + 1) if k + 1 < NCHUNK else []
        for cp in pend:
            cp.wait()
        pend = nxt
        if len(outcps) == 2:
            outcps.pop(0).wait()

        def ebody(e, carry):
            ev = jnp.full((16,), e, jnp.int32) + k * C
            w0 = plsc.load_gather(wbuf, [ev])
            w1 = plsc.load_gather(wbuf, [ev + PER_TILE])
            w2 = plsc.load_gather(wbuf, [ev + 2 * PER_TILE])
            se = e + s * C
            r2l = rows2lo[se, pl.ds(16, 16)]
            r2h = rows2hi[se, pl.ds(16, 16)]
            v2 = r2l + w2 * (r2h - r2l)
            r0l = rows0lo[se, pl.ds(0, 16)]
            r0h = rows0hi[se, pl.ds(0, 16)]
            v0 = r0l + w0 * (r0h - r0l)
            ul0 = jnp.zeros((R,), jnp.float32)
            ul1 = jnp.zeros((R,), jnp.float32)
            uh0 = jnp.zeros((R,), jnp.float32)
            uh1 = jnp.zeros((R,), jnp.float32)
            for a in range(R):
                v0a = v0[a]
                ml = rows1lo[se, pl.ds(a * 16, 16)]
                mh = rows1hi[se, pl.ds(a * 16, 16)]
                if a % 2 == 0:
                    ul0 = ul0 + v0a * ml
                    uh0 = uh0 + v0a * mh
                else:
                    ul1 = ul1 + v0a * ml
                    uh1 = uh1 + v0a * mh
            ul = ul0 + ul1
            uh = uh0 + uh1
            u = ul + w1 * (uh - ul)
            ans = jnp.sum(u * v2)
            plsc.store_scatter(outv, [jnp.full((16,), se, jnp.int32)],
                               jnp.full((16,), ans, jnp.float32), mask=lane0)
            return carry

        lax.fori_loop(0, C, ebody, 0)
        outcps.append(pltpu.async_copy(outv.at[pl.ds(s * C, C)],
                                       out.at[pl.ds(base0 + k * C, C)], sos[s]))
    for cp in outcps:
        cp.wait()


def kernel(x, core0, core1, core2):
    xT = x.T.reshape(3 * B)
    big = core1.transpose(1, 0, 2).reshape(N0, 2 * 128)
    t0 = core0.transpose(0, 2, 1).reshape(R, N0)
    t2 = core2.transpose(0, 2, 1).reshape(R, N0)
    small = _small_build(t0, t2)
    return _tt_sc(xT, big, small)


# speedup vs baseline: 1.0204x; 1.0204x over previous
import functools

import jax
import jax.numpy as jnp
from jax import lax
from jax.experimental import pallas as pl
from jax.experimental.pallas import tpu as pltpu
from jax.experimental.pallas import tpu_sc as plsc

N0 = 100000
R = 16
B = 16384
NC = 2
NSUB = 16
NW = NC * NSUB
PER_TILE = B // NW
C = 32
NCHUNK = PER_TILE // C

_mesh = plsc.VectorSubcoreMesh(
    core_axis_name="c", subcore_axis_name="s", num_cores=NC, num_subcores=NSUB
)


JCH = 782


@functools.partial(
    pl.kernel,
    out_type=jax.ShapeDtypeStruct((N0, 128), jnp.float32),
    mesh=_mesh,
    compiler_params=pltpu.CompilerParams(
        needs_layout_passes=False, use_tc_tiling_on_sc=True,
        disable_bounds_checks=True),
    scratch_types=[
        pltpu.VMEM((2 * 16, 128), jnp.float32),
        pltpu.VMEM((2 * 16, 128), jnp.float32),
        pltpu.VMEM((2 * 128, 128), jnp.float32),
        pltpu.SemaphoreType.DMA,
        pltpu.SemaphoreType.DMA,
        pltpu.SemaphoreType.DMA,
        pltpu.SemaphoreType.DMA,
    ],
)
def _small_build(t0, t2, out, b0, b2, orows, sin0, sin1, sout0, sout1):
    wid = lax.axis_index("s") * NC + lax.axis_index("c")
    lane16 = lax.iota(jnp.int32, 16)
    NFULL = JCH - 1
    NIT = (NFULL + NW - 1) // NW
    sins = (sin0, sin1)
    souts = (sout0, sout1)

    def ch_of(it):
        return jnp.minimum(it * NW + wid, NFULL - 1)

    def fire(it):
        s = it % 2
        jb = ch_of(it) * 128
        return [
            pltpu.async_copy(t0.at[pl.ds(0, 16), pl.ds(jb, 128)],
                             b0.at[pl.ds(s * 16, 16)], sins[s]),
            pltpu.async_copy(t2.at[pl.ds(0, 16), pl.ds(jb, 128)],
                             b2.at[pl.ds(s * 16, 16)], sins[s]),
        ]

    def transpose_rows(s, nrows):
        def jrow(jl, c):
            jv = jnp.full((16,), jl, jnp.int32)
            orows[jl + s * 128, pl.ds(0, 16)] = plsc.load_gather(
                b0, [lane16 + s * 16, jv])
            orows[jl + s * 128, pl.ds(16, 16)] = plsc.load_gather(
                b2, [lane16 + s * 16, jv])
            return c
        lax.fori_loop(0, nrows, jrow, 0)

    pend = fire(0)
    outcps = []
    for it in range(NIT):
        s = it % 2
        nxt = fire(it + 1) if it + 1 < NIT else []
        for cp in pend:
            cp.wait()
        pend = nxt
        if len(outcps) == 2:
            outcps.pop(0).wait()
        transpose_rows(s, 128)
        outcps.append(pltpu.async_copy(
            orows.at[pl.ds(s * 128, 128)],
            out.at[pl.ds(ch_of(it) * 128, 128)], souts[s]))
    for cp in outcps:
        cp.wait()

    @pl.when(wid == NW - 1)
    def _tail():
        jb = pl.multiple_of(NFULL * 128 + (wid - (NW - 1)), 128)
        cp0 = pltpu.async_copy(t0.at[pl.ds(0, 16), pl.ds(jb, 128)],
                               b0.at[pl.ds(0, 16)], sin0)
        cp1 = pltpu.async_copy(t2.at[pl.ds(0, 16), pl.ds(jb, 128)],
                               b2.at[pl.ds(0, 16)], sin0)
        cp0.wait()
        cp1.wait()
        transpose_rows(0, 32)
        pltpu.async_copy(orows.at[pl.ds(0, 32)],
                         out.at[pl.ds(jb, 32)], sout0).wait()


@functools.partial(
    pl.kernel,
    out_type=jax.ShapeDtypeStruct((B,), jnp.float32),
    mesh=_mesh,
    compiler_params=pltpu.CompilerParams(
        needs_layout_passes=False, use_tc_tiling_on_sc=True),
    scratch_types=[
        pltpu.VMEM((3 * PER_TILE,), jnp.float32),
        pltpu.VMEM((3 * PER_TILE,), jnp.float32),
        pltpu.VMEM((PER_TILE,), jnp.int32),
        pltpu.VMEM((PER_TILE,), jnp.int32),
        pltpu.VMEM((PER_TILE,), jnp.int32),
        pltpu.VMEM((PER_TILE,), jnp.int32),
        pltpu.VMEM((PER_TILE,), jnp.int32),
        pltpu.VMEM((PER_TILE,), jnp.int32),
        pltpu.VMEM((3 * C, 128), jnp.float32),
        pltpu.VMEM((3 * C, 128), jnp.float32),
        pltpu.VMEM((3 * C, 128), jnp.float32),
        pltpu.VMEM((3 * C, 128), jnp.float32),
        pltpu.VMEM((3 * C, 256), jnp.float32),
        pltpu.VMEM((3 * C, 256), jnp.float32),
        pltpu.VMEM((3 * C,), jnp.float32),
        pltpu.SemaphoreType.DMA,
        pltpu.SemaphoreType.DMA,
        pltpu.SemaphoreType.DMA,
        pltpu.SemaphoreType.DMA,
        pltpu.SemaphoreType.DMA,
        pltpu.SemaphoreType.DMA,
    ],
)
def _tt_sc(xT, big, small, out, xbuf, wbuf, jlo0, jhi0, jlo1, jhi1,
           jlo2, jhi2, rows0lo, rows0hi, rows2lo, rows2hi,
           rows1lo, rows1hi, outv, sg0, sg1, sg2, so0, so1, so2):
    sgs = (sg0, sg1, sg2)
    sos = (so0, so1, so2)
    wid = lax.axis_index("s") * NC + lax.axis_index("c")
    base0 = wid * PER_TILE

    xcp = [pltpu.async_copy(xT.at[pl.ds(i * B + base0, PER_TILE)],
                            xbuf.at[pl.ds(i * PER_TILE, PER_TILE)], sg0)
           for i in range(3)]
    for cp in xcp:
        cp.wait()

    for i in range(3):
        jlo_ref = (jlo0, jlo1, jlo2)[i]
        jhi_ref = (jhi0, jhi1, jhi2)[i]
        for t in range(PER_TILE // 16):
            sl = pl.ds(t * 16, 16)
            xv = xbuf[pl.ds(i * PER_TILE + t * 16, 16)]
            xr = (xv + 1.0) * (0.5 * (N0 - 1))
            xr = jnp.minimum(jnp.maximum(xr, 0.0), float(N0 - 1))
            jlo = xr.astype(jnp.int32)
            w = xr - jlo.astype(jnp.float32)
            jhi = jnp.where(w > 0.0, jlo + 1, jlo)
            wbuf[pl.ds(i * PER_TILE + t * 16, 16)] = w
            jlo_ref[sl] = jlo
            jhi_ref[sl] = jhi

    def fire(k):
        sg = sgs[k % 3]
        ssl = pl.ds((k % 3) * C, C)
        ksl = pl.ds(k * C, C)
        return [
            pltpu.async_copy(small.at[jlo0.at[ksl]], rows0lo.at[ssl], sg),
            pltpu.async_copy(small.at[jhi0.at[ksl]], rows0hi.at[ssl], sg),
            pltpu.async_copy(small.at[jlo2.at[ksl]], rows2lo.at[ssl], sg),
            pltpu.async_copy(small.at[jhi2.at[ksl]], rows2hi.at[ssl], sg),
            pltpu.async_copy(big.at[jlo1.at[ksl]], rows1lo.at[ssl], sg),
            pltpu.async_copy(big.at[jhi1.at[ksl]], rows1hi.at[ssl], sg),
        ]

    lane = lax.iota(jnp.int32, 16)
    lane0 = lane == 0
    outcps = []
    pend = [fire(0), fire(1)]
    for k in range(NCHUNK):
        s = k % 3
        if k + 2 < NCHUNK:
            pend.append(fire(k + 2))
        for cp in pend.pop(0):
            cp.wait()
        if len(outcps) == 3:
            outcps.pop(0).wait()

        def ebody(e, carry):
            ev = jnp.full((16,), e, jnp.int32) + k * C
            w0 = plsc.load_gather(wbuf, [ev])
            w1 = plsc.load_gather(wbuf, [ev + PER_TILE])
            w2 = plsc.load_gather(wbuf, [ev + 2 * PER_TILE])
            se = e + s * C
            r2l = rows2lo[se, pl.ds(16, 16)]
            r2h = rows2hi[se, pl.ds(16, 16)]
            v2 = r2l + w2 * (r2h - r2l)
            r0l = rows0lo[se, pl.ds(0, 16)]
            r0h = rows0hi[se, pl.ds(0, 16)]
            v0 = r0l + w0 * (r0h - r0l)
            ul0 = jnp.zeros((R,), jnp.float32)
            ul1 = jnp.zeros((R,), jnp.float32)
            uh0 = jnp.zeros((R,), jnp.float32)
            uh1 = jnp.zeros((R,), jnp.float32)
            for a in range(R):
                v0a = v0[a]
                ml = rows1lo[se, pl.ds(a * 16, 16)]
                mh = rows1hi[se, pl.ds(a * 16, 16)]
                if a % 2 == 0:
                    ul0 = ul0 + v0a * ml
                    uh0 = uh0 + v0a * mh
                else:
                    ul1 = ul1 + v0a * ml
                    uh1 = uh1 + v0a * mh
            ul = ul0 + ul1
            uh = uh0 + uh1
            u = ul + w1 * (uh - ul)
            ans = jnp.sum(u * v2)
            plsc.store_scatter(outv, [jnp.full((16,), se, jnp.int32)],
                               jnp.full((16,), ans, jnp.float32), mask=lane0)
            return carry

        lax.fori_loop(0, C, ebody, 0)
        outcps.append(pltpu.async_copy(outv.at[pl.ds(s * C, C)],
                                       out.at[pl.ds(base0 + k * C, C)], sos[s]))
    for cp in outcps:
        cp.wait()


def kernel(x, core0, core1, core2):
    xT = x.T.reshape(3 * B)
    big = core1.transpose(1, 0, 2).reshape(N0, 2 * 128)
    t0 = core0.transpose(0, 2, 1).reshape(R, N0)
    t2 = core2.transpose(0, 2, 1).reshape(R, N0)
    small = _small_build(t0, t2)
    return _tt_sc(xT, big, small)
